# baseline (speedup 1.0000x reference)
import jax
import jax.numpy as jnp
from jax.experimental import pallas as pl
from jax.experimental.pallas import tpu as pltpu

N, K, D = 16384, 1024, 64
BN = 4096
GRID = N // BN


def _kmeans_body(x_ref, c_ref, out_ref, acc_ref, cnt_ref):
    i = pl.program_id(0)

    @pl.when(i == 0)
    def _init():
        acc_ref[...] = jnp.zeros_like(acc_ref)
        cnt_ref[...] = jnp.zeros_like(cnt_ref)

    x = x_ref[...]
    c = c_ref[...]
    cross = jax.lax.dot_general(
        x, c, (((1,), (1,)), ((), ())), preferred_element_type=jnp.float32
    )
    x_sq = jnp.sum(x * x, axis=1, keepdims=True)
    c_sq = jnp.sum(c * c, axis=1)[None, :]
    dist = x_sq - 2.0 * cross + c_sq

    kiota = jax.lax.broadcasted_iota(jnp.int32, (BN, K), 1)
    idx = jnp.argmin(dist, axis=1)[:, None]
    onehot = (kiota == idx).astype(jnp.float32)

    acc_ref[...] += jax.lax.dot_general(
        onehot, x, (((0,), (0,)), ((), ())), preferred_element_type=jnp.float32
    )
    ones = jnp.ones((BN, 1), jnp.bfloat16)
    cnt_ref[...] += jax.lax.dot_general(
        onehot.astype(jnp.bfloat16), ones, (((0,), (0,)), ((), ())),
        preferred_element_type=jnp.float32,
    )

    @pl.when(i == GRID - 1)
    def _finish():
        out_ref[...] = acc_ref[...] / jnp.maximum(cnt_ref[...], 1.0)


def kernel(input_x, input_centroids):
    return pl.pallas_call(
        _kmeans_body,
        grid=(GRID,),
        in_specs=[
            pl.BlockSpec((BN, D), lambda i: (i, 0)),
            pl.BlockSpec((K, D), lambda i: (0, 0)),
        ],
        out_specs=pl.BlockSpec((K, D), lambda i: (0, 0)),
        out_shape=jax.ShapeDtypeStruct((K, D), jnp.float32),
        scratch_shapes=[
            pltpu.VMEM((K, D), jnp.float32),
            pltpu.VMEM((K, 1), jnp.float32),
        ],
    )(input_x, input_centroids)


# speedup vs baseline: 1.0308x; 1.0308x over previous
import jax
import jax.numpy as jnp
from jax.experimental import pallas as pl
from jax.experimental.pallas import tpu as pltpu

N, K, D = 16384, 1024, 64
BN = 4096
GRID = N // BN


def _kmeans_body(x_ref, c2_ref, out_ref, acc_ref, cnt_ref):
    i = pl.program_id(0)

    @pl.when(i == 0)
    def _init():
        acc_ref[...] = jnp.zeros_like(acc_ref)
        cnt_ref[...] = jnp.zeros_like(cnt_ref)

    x = x_ref[...]
    c2 = c2_ref[...]
    cross2 = jax.lax.dot_general(
        x, c2, (((1,), (1,)), ((), ())), preferred_element_type=jnp.float32
    )
    x_sq = jnp.sum(x * x, axis=1, keepdims=True)
    ch = c2 * 0.5
    c_sq = jnp.sum(ch * ch, axis=1)[None, :]
    dist = x_sq - cross2 + c_sq

    kiota = jax.lax.broadcasted_iota(jnp.int32, (BN, K), 1)
    idx = jnp.argmin(dist, axis=1)[:, None]
    onehot = (kiota == idx).astype(jnp.float32)

    dn = (((0,), (0,)), ((), ()))
    acc_ref[...] += jax.lax.dot_general(
        onehot, x, dn, preferred_element_type=jnp.float32
    )
    ones = jnp.ones((BN, 1), jnp.float32)
    cnt_ref[...] += jax.lax.dot_general(
        onehot, ones, dn, preferred_element_type=jnp.float32
    )

    @pl.when(i == GRID - 1)
    def _finish():
        out_ref[...] = acc_ref[...] / jnp.maximum(cnt_ref[...], 1.0)


def kernel(input_x, input_centroids):
    return pl.pallas_call(
        _kmeans_body,
        grid=(GRID,),
        in_specs=[
            pl.BlockSpec((BN, D), lambda i: (i, 0)),
            pl.BlockSpec((K, D), lambda i: (0, 0)),
        ],
        out_specs=pl.BlockSpec((K, D), lambda i: (0, 0)),
        out_shape=jax.ShapeDtypeStruct((K, D), jnp.float32),
        scratch_shapes=[
            pltpu.VMEM((K, D), jnp.float32),
            pltpu.VMEM((K, 1), jnp.float32),
        ],
    )(input_x, input_centroids + input_centroids)


# speedup vs baseline: 1.5471x; 1.5008x over previous
import jax
import jax.numpy as jnp
from jax.experimental import pallas as pl
from jax.experimental.pallas import tpu as pltpu

N, K, D = 16384, 1024, 64
BN = 4096
GRID = N // BN


def _kmeans_body(xt_ref, ct2_ref, out_ref, acc_ref, cnt_ref):
    i = pl.program_id(0)

    @pl.when(i == 0)
    def _init():
        acc_ref[...] = jnp.zeros_like(acc_ref)
        cnt_ref[...] = jnp.zeros_like(cnt_ref)

    xt = xt_ref[...]
    ct2 = ct2_ref[...]
    cross2 = jax.lax.dot_general(
        xt, ct2, (((0,), (0,)), ((), ())), preferred_element_type=jnp.float32
    )
    x_sq = jnp.transpose(jnp.sum(xt * xt, axis=0, keepdims=True))
    ch = ct2 * 0.5
    c_sq = jnp.sum(ch * ch, axis=0, keepdims=True)
    dist = x_sq - cross2 + c_sq

    kiota = jax.lax.broadcasted_iota(jnp.int32, (BN, K), 1)
    idx = jnp.argmin(dist, axis=1)[:, None]
    onehot = (kiota == idx).astype(jnp.float32)

    dn = (((1,), (0,)), ((), ()))
    acc_ref[...] += jax.lax.dot_general(
        xt, onehot, dn, preferred_element_type=jnp.float32
    )
    ones = jnp.ones((1, BN), jnp.float32)
    cnt_ref[...] += jax.lax.dot_general(
        ones, onehot, dn, preferred_element_type=jnp.float32
    )

    @pl.when(i == GRID - 1)
    def _finish():
        out_ref[...] = acc_ref[...] / jnp.maximum(cnt_ref[...], 1.0)


def kernel(input_x, input_centroids):
    c2t = (input_centroids + input_centroids).T
    out_t = pl.pallas_call(
        _kmeans_body,
        grid=(GRID,),
        in_specs=[
            pl.BlockSpec((D, BN), lambda i: (0, i)),
            pl.BlockSpec((D, K), lambda i: (0, 0)),
        ],
        out_specs=pl.BlockSpec((D, K), lambda i: (0, 0)),
        out_shape=jax.ShapeDtypeStruct((D, K), jnp.float32),
        scratch_shapes=[
            pltpu.VMEM((D, K), jnp.float32),
            pltpu.VMEM((1, K), jnp.float32),
        ],
    )(input_x.T, c2t)
    return out_t.T
